# baseline (speedup 1.0000x reference)
import jax
import jax.numpy as jnp
from jax.experimental import pallas as pl
from jax.experimental.pallas import tpu as pltpu

_NCHUNK = 8


def _dma_kernel(table_ref, out_ref, *sems):
    rows = out_ref.shape[0] // _NCHUNK
    copies = []
    for i in range(_NCHUNK):
        c = pltpu.make_async_copy(
            table_ref.at[pl.ds(i * rows, rows)],
            out_ref.at[pl.ds(i * rows, rows)],
            sems[i],
        )
        c.start()
        copies.append(c)
    for c in copies:
        c.wait()


def kernel(inputs, pos_table):
    seq_len = inputs.shape[-1]
    _, embed_dim = pos_table.shape
    return pl.pallas_call(
        _dma_kernel,
        in_specs=[pl.BlockSpec(memory_space=pltpu.MemorySpace.HBM)],
        out_specs=pl.BlockSpec(memory_space=pltpu.MemorySpace.HBM),
        scratch_shapes=[pltpu.SemaphoreType.DMA] * _NCHUNK,
        out_shape=jax.ShapeDtypeStruct((seq_len, embed_dim), pos_table.dtype),
    )(pos_table)
